# Initial kernel scaffold; baseline (speedup 1.0000x reference)
import jax
import jax.numpy as jnp
from jax.experimental import pallas as pl


def kernel(x, edge_index, W1, b1, W2, b2):
    raise NotImplementedError("write your pallas kernel here")


# speedup vs baseline: 12.5959x; 12.5959x over previous
import functools

import jax
import jax.numpy as jnp
from jax import lax
from jax.experimental import pallas as pl
from jax.experimental.pallas import tpu as pltpu
from jax.experimental.pallas import tpu_sc as plsc

N = 10000
E = 320000
D1 = 128
D2 = 40
NC = 2
NS = 16
NW = NC * NS
EPW = E // NW
CH = 128
NFULL = EPW // CH
TAIL = EPW - NFULL * CH
RPT = 624
REM = N - NS * RPT


@functools.cache
def _make_agg():
    mesh = plsc.VectorSubcoreMesh(
        core_axis_name="c", subcore_axis_name="s", num_cores=NC, num_subcores=NS
    )

    @functools.partial(
        pl.kernel,
        out_type=jax.ShapeDtypeStruct((NC, N, D1), jnp.float32),
        mesh=mesh,
        scratch_types=[
            pltpu.VMEM_SHARED((N, D1), jnp.float32),
            pltpu.VMEM((CH, D1), jnp.float32),
            pltpu.VMEM((CH,), jnp.int32),
            pltpu.VMEM((CH,), jnp.int32),
            pltpu.VMEM((TAIL, D1), jnp.float32),
            pltpu.VMEM((TAIL,), jnp.int32),
            pltpu.VMEM((TAIL,), jnp.int32),
            pltpu.SemaphoreType.DMA,
        ],
    )
    def _agg(z_hbm, src_hbm, dst_hbm, out_hbm,
             acc_sh, rows_v, srci, dsti, rows_t, srci_t, dsti_t, sem):
        c = lax.axis_index("c")
        s = lax.axis_index("s")
        wid = c * NS + s

        pltpu.sync_copy(
            z_hbm.at[pl.ds(s * RPT, RPT)], acc_sh.at[pl.ds(s * RPT, RPT)]
        )

        @pl.when(s == 0)
        def _():
            pltpu.sync_copy(
                z_hbm.at[pl.ds(NS * RPT, REM)], acc_sh.at[pl.ds(NS * RPT, REM)]
            )

        plsc.subcore_barrier()

        base = wid * EPW

        def body(i, carry):
            off = base + i * CH
            pltpu.sync_copy(src_hbm.at[pl.ds(off, CH)], srci)
            pltpu.sync_copy(dst_hbm.at[pl.ds(off, CH)], dsti)
            pltpu.async_copy(z_hbm.at[srci], rows_v, sem).wait()
            pltpu.sync_copy(rows_v, acc_sh.at[dsti], add=True)
            return carry

        lax.fori_loop(0, NFULL, body, 0)
        toff = base + NFULL * CH
        pltpu.sync_copy(src_hbm.at[pl.ds(toff, TAIL)], srci_t)
        pltpu.sync_copy(dst_hbm.at[pl.ds(toff, TAIL)], dsti_t)
        pltpu.async_copy(z_hbm.at[srci_t], rows_t, sem).wait()
        pltpu.sync_copy(rows_t, acc_sh.at[dsti_t], add=True)

        plsc.subcore_barrier()
        pltpu.sync_copy(
            acc_sh.at[pl.ds(s * RPT, RPT)], out_hbm.at[c, pl.ds(s * RPT, RPT)]
        )

        @pl.when(s == 0)
        def _():
            pltpu.sync_copy(
                acc_sh.at[pl.ds(NS * RPT, REM)],
                out_hbm.at[c, pl.ds(NS * RPT, REM)],
            )

    return _agg


_RB = 1000
_GRID = N // _RB


def _dinv_block(ones_ref):
    deg = ones_ref[0, :, 0:1] + ones_ref[1, :, 0:1] - 1.0
    return lax.rsqrt(jnp.maximum(deg, 1e-12))


def _tc1_body(ones_ref, x_ref, w_ref, z_ref):
    dinv = _dinv_block(ones_ref)
    z_ref[...] = dinv * jnp.dot(x_ref[...], w_ref[...],
                                preferred_element_type=jnp.float32)


def _tc1(acc_ones, x, W1):
    return pl.pallas_call(
        _tc1_body,
        out_shape=jax.ShapeDtypeStruct((N, D1), jnp.float32),
        grid=(_GRID,),
        in_specs=[
            pl.BlockSpec((NC, _RB, D1), lambda i: (0, i, 0)),
            pl.BlockSpec((_RB, D1), lambda i: (i, 0)),
            pl.BlockSpec((D1, D1), lambda i: (0, 0)),
        ],
        out_specs=pl.BlockSpec((_RB, D1), lambda i: (i, 0)),
    )(acc_ones, x, W1)


def _tc2_body(acc_ref, z1_ref, ones_ref, b1_ref, w2_ref, z2_ref):
    dinv = _dinv_block(ones_ref)
    agg = acc_ref[0] + acc_ref[1] - z1_ref[...]
    h = jnp.maximum(dinv * agg + b1_ref[...], 0.0)
    z2_ref[...] = dinv * jnp.dot(h, w2_ref[...],
                                 preferred_element_type=jnp.float32)


def _tc2(acc1, z1, acc_ones, b1r, W2p):
    return pl.pallas_call(
        _tc2_body,
        out_shape=jax.ShapeDtypeStruct((N, D1), jnp.float32),
        grid=(_GRID,),
        in_specs=[
            pl.BlockSpec((NC, _RB, D1), lambda i: (0, i, 0)),
            pl.BlockSpec((_RB, D1), lambda i: (i, 0)),
            pl.BlockSpec((NC, _RB, D1), lambda i: (0, i, 0)),
            pl.BlockSpec((1, D1), lambda i: (0, 0)),
            pl.BlockSpec((D1, D1), lambda i: (0, 0)),
        ],
        out_specs=pl.BlockSpec((_RB, D1), lambda i: (i, 0)),
    )(acc1, z1, acc_ones, b1r, W2p)


def _tc3_body(acc_ref, z2_ref, ones_ref, b2_ref, o_ref):
    dinv = _dinv_block(ones_ref)
    logits = dinv * (acc_ref[0] + acc_ref[1] - z2_ref[...]) + b2_ref[...]
    col = lax.broadcasted_iota(jnp.int32, (_RB, D1), 1)
    logits = jnp.where(col < D2, logits, jnp.full_like(logits, -1e30))
    m = jnp.max(logits, axis=1, keepdims=True)
    lse = jnp.log(jnp.sum(jnp.exp(logits - m), axis=1, keepdims=True))
    o_ref[...] = (logits - m - lse)[:, :D2]


def _tc3(acc2, z2, acc_ones, b2p):
    return pl.pallas_call(
        _tc3_body,
        out_shape=jax.ShapeDtypeStruct((N, D2), jnp.float32),
        grid=(_GRID,),
        in_specs=[
            pl.BlockSpec((NC, _RB, D1), lambda i: (0, i, 0)),
            pl.BlockSpec((_RB, D1), lambda i: (i, 0)),
            pl.BlockSpec((NC, _RB, D1), lambda i: (0, i, 0)),
            pl.BlockSpec((1, D1), lambda i: (0, 0)),
        ],
        out_specs=pl.BlockSpec((_RB, D2), lambda i: (i, 0)),
    )(acc2, z2, acc_ones, b2p)


def kernel(x, edge_index, W1, b1, W2, b2):
    src = edge_index[0].astype(jnp.int32)
    dst = edge_index[1].astype(jnp.int32)
    W2p = jnp.pad(W2, ((0, 0), (0, D1 - D2)))
    b1r = b1.reshape(1, D1)
    b2p = jnp.pad(b2, (0, D1 - D2)).reshape(1, D1)
    ones_m = jnp.ones((N, D1), jnp.float32)

    agg = _make_agg()
    acc_ones = agg(ones_m, src, dst)
    z1 = _tc1(acc_ones, x, W1)
    acc1 = agg(z1, src, dst)
    z2 = _tc2(acc1, z1, acc_ones, b1r, W2p)
    acc2 = agg(z2, src, dst)
    return _tc3(acc2, z2, acc_ones, b2p)


# speedup vs baseline: 17.0681x; 1.3551x over previous
import functools

import jax
import jax.numpy as jnp
from jax import lax
from jax.experimental import pallas as pl
from jax.experimental.pallas import tpu as pltpu
from jax.experimental.pallas import tpu_sc as plsc

N = 10000
E = 320000
D1 = 128
D2 = 40
NC = 2
NS = 16
NW = NC * NS
EPW = E // NW
CH = 128
NFULL = EPW // CH
TAIL = EPW - NFULL * CH
RPT = 624
REM = N - NS * RPT


@functools.cache
def _make_agg():
    mesh = plsc.VectorSubcoreMesh(
        core_axis_name="c", subcore_axis_name="s", num_cores=NC, num_subcores=NS
    )

    @functools.partial(
        pl.kernel,
        out_type=jax.ShapeDtypeStruct((NC, N, D1), jnp.float32),
        mesh=mesh,
        scratch_types=[
            pltpu.VMEM_SHARED((N, D1), jnp.float32),
            pltpu.VMEM((CH, D1), jnp.float32),
            pltpu.VMEM((CH, D1), jnp.float32),
            pltpu.VMEM((CH,), jnp.int32),
            pltpu.VMEM((CH,), jnp.int32),
            pltpu.VMEM((CH,), jnp.int32),
            pltpu.VMEM((CH,), jnp.int32),
            pltpu.VMEM((TAIL, D1), jnp.float32),
            pltpu.VMEM((TAIL,), jnp.int32),
            pltpu.VMEM((TAIL,), jnp.int32),
            pltpu.SemaphoreType.DMA,
            pltpu.SemaphoreType.DMA,
        ],
    )
    def _agg(z_hbm, src_hbm, dst_hbm, out_hbm,
             acc_sh, rows_v, rows_w, srci, dsti, srcj, dstj,
             rows_t, srci_t, dsti_t, sem, sem2):
        c = lax.axis_index("c")
        s = lax.axis_index("s")
        wid = c * NS + s

        pltpu.sync_copy(
            z_hbm.at[pl.ds(s * RPT, RPT)], acc_sh.at[pl.ds(s * RPT, RPT)]
        )

        @pl.when(s == 0)
        def _():
            pltpu.sync_copy(
                z_hbm.at[pl.ds(NS * RPT, REM)], acc_sh.at[pl.ds(NS * RPT, REM)]
            )

        plsc.subcore_barrier()

        base = wid * EPW

        def body(i, carry):
            off0 = base + 2 * i * CH
            off1 = off0 + CH
            pltpu.sync_copy(src_hbm.at[pl.ds(off0, CH)], srci)
            pltpu.sync_copy(dst_hbm.at[pl.ds(off0, CH)], dsti)
            g0 = pltpu.async_copy(z_hbm.at[srci], rows_v, sem)
            pltpu.sync_copy(src_hbm.at[pl.ds(off1, CH)], srcj)
            pltpu.sync_copy(dst_hbm.at[pl.ds(off1, CH)], dstj)
            g1 = pltpu.async_copy(z_hbm.at[srcj], rows_w, sem2)
            g0.wait()
            pltpu.sync_copy(rows_v, acc_sh.at[dsti], add=True)
            g1.wait()
            pltpu.sync_copy(rows_w, acc_sh.at[dstj], add=True)
            return carry

        lax.fori_loop(0, NFULL // 2, body, 0)
        toff = base + NFULL * CH
        pltpu.sync_copy(src_hbm.at[pl.ds(toff, TAIL)], srci_t)
        pltpu.sync_copy(dst_hbm.at[pl.ds(toff, TAIL)], dsti_t)
        pltpu.async_copy(z_hbm.at[srci_t], rows_t, sem).wait()
        pltpu.sync_copy(rows_t, acc_sh.at[dsti_t], add=True)

        plsc.subcore_barrier()
        pltpu.sync_copy(
            acc_sh.at[pl.ds(s * RPT, RPT)], out_hbm.at[c, pl.ds(s * RPT, RPT)]
        )

        @pl.when(s == 0)
        def _():
            pltpu.sync_copy(
                acc_sh.at[pl.ds(NS * RPT, REM)],
                out_hbm.at[c, pl.ds(NS * RPT, REM)],
            )

    return _agg


_RB = 1000
_GRID = N // _RB


def _dinv_block(ones_ref):
    deg = ones_ref[0, :, 0:1] + ones_ref[1, :, 0:1] - 1.0
    return lax.rsqrt(jnp.maximum(deg, 1e-12))


def _tc1_body(ones_ref, x_ref, w_ref, z_ref):
    dinv = _dinv_block(ones_ref)
    z_ref[...] = dinv * jnp.dot(x_ref[...], w_ref[...],
                                preferred_element_type=jnp.float32)


def _tc1(acc_ones, x, W1):
    return pl.pallas_call(
        _tc1_body,
        out_shape=jax.ShapeDtypeStruct((N, D1), jnp.float32),
        grid=(_GRID,),
        in_specs=[
            pl.BlockSpec((NC, _RB, D1), lambda i: (0, i, 0)),
            pl.BlockSpec((_RB, D1), lambda i: (i, 0)),
            pl.BlockSpec((D1, D1), lambda i: (0, 0)),
        ],
        out_specs=pl.BlockSpec((_RB, D1), lambda i: (i, 0)),
    )(acc_ones, x, W1)


def _tc2_body(acc_ref, z1_ref, ones_ref, b1_ref, w2_ref, z2_ref):
    dinv = _dinv_block(ones_ref)
    agg = acc_ref[0] + acc_ref[1] - z1_ref[...]
    h = jnp.maximum(dinv * agg + b1_ref[...], 0.0)
    z2_ref[...] = dinv * jnp.dot(h, w2_ref[...],
                                 preferred_element_type=jnp.float32)


def _tc2(acc1, z1, acc_ones, b1r, W2p):
    return pl.pallas_call(
        _tc2_body,
        out_shape=jax.ShapeDtypeStruct((N, D1), jnp.float32),
        grid=(_GRID,),
        in_specs=[
            pl.BlockSpec((NC, _RB, D1), lambda i: (0, i, 0)),
            pl.BlockSpec((_RB, D1), lambda i: (i, 0)),
            pl.BlockSpec((NC, _RB, D1), lambda i: (0, i, 0)),
            pl.BlockSpec((1, D1), lambda i: (0, 0)),
            pl.BlockSpec((D1, D1), lambda i: (0, 0)),
        ],
        out_specs=pl.BlockSpec((_RB, D1), lambda i: (i, 0)),
    )(acc1, z1, acc_ones, b1r, W2p)


def _tc3_body(acc_ref, z2_ref, ones_ref, b2_ref, o_ref):
    dinv = _dinv_block(ones_ref)
    logits = dinv * (acc_ref[0] + acc_ref[1] - z2_ref[...]) + b2_ref[...]
    col = lax.broadcasted_iota(jnp.int32, (_RB, D1), 1)
    logits = jnp.where(col < D2, logits, jnp.full_like(logits, -1e30))
    m = jnp.max(logits, axis=1, keepdims=True)
    lse = jnp.log(jnp.sum(jnp.exp(logits - m), axis=1, keepdims=True))
    o_ref[...] = (logits - m - lse)[:, :D2]


def _tc3(acc2, z2, acc_ones, b2p):
    return pl.pallas_call(
        _tc3_body,
        out_shape=jax.ShapeDtypeStruct((N, D2), jnp.float32),
        grid=(_GRID,),
        in_specs=[
            pl.BlockSpec((NC, _RB, D1), lambda i: (0, i, 0)),
            pl.BlockSpec((_RB, D1), lambda i: (i, 0)),
            pl.BlockSpec((NC, _RB, D1), lambda i: (0, i, 0)),
            pl.BlockSpec((1, D1), lambda i: (0, 0)),
        ],
        out_specs=pl.BlockSpec((_RB, D2), lambda i: (i, 0)),
    )(acc2, z2, acc_ones, b2p)


def kernel(x, edge_index, W1, b1, W2, b2):
    src = edge_index[0].astype(jnp.int32)
    dst = edge_index[1].astype(jnp.int32)
    W2p = jnp.pad(W2, ((0, 0), (0, D1 - D2)))
    b1r = b1.reshape(1, D1)
    b2p = jnp.pad(b2, (0, D1 - D2)).reshape(1, D1)
    ones_m = jnp.ones((N, D1), jnp.float32)

    agg = _make_agg()
    acc_ones = agg(ones_m, src, dst)
    z1 = _tc1(acc_ones, x, W1)
    acc1 = agg(z1, src, dst)
    z2 = _tc2(acc1, z1, acc_ones, b1r, W2p)
    acc2 = agg(z2, src, dst)
    return _tc3(acc2, z2, acc_ones, b2p)
